# Initial kernel scaffold; baseline (speedup 1.0000x reference)
import jax
import jax.numpy as jnp
from jax.experimental import pallas as pl


def kernel(imgs, xform_params):
    raise NotImplementedError("write your pallas kernel here")


# speedup vs baseline: 428.7269x; 428.7269x over previous
import functools

import jax
import jax.numpy as jnp
from jax.experimental import pallas as pl
from jax.experimental.pallas import tpu as pltpu
from jax.experimental.pallas import tpu_sc as plsc

_LANES = 16
_CHUNK = 16384
_UNROLL = 4


def _make_sc_call(P, PIX, R):
    nch = PIX // _CHUNK
    scale = jnp.float32(R - 1)
    mesh = plsc.VectorSubcoreMesh(core_axis_name="c", subcore_axis_name="s")

    @functools.partial(
        pl.kernel,
        out_type=jax.ShapeDtypeStruct((P, PIX), jnp.float32),
        mesh=mesh,
        compiler_params=pltpu.CompilerParams(needs_layout_passes=False),
    )
    def run(imgs_hbm, lut_hbm, out_hbm):
        def body(in_v, lut_v, out_v):
            zero = jnp.zeros((_LANES,), jnp.int32)

            @pl.loop(0, _CHUNK, step=_LANES * _UNROLL)
            def _(c):
                for u in range(_UNROLL):
                    x = in_v[0, pl.ds(c + u * _LANES, _LANES)]
                    s = x * scale
                    sc = jnp.minimum(jnp.maximum(s, 0.0), scale)
                    i0 = sc.astype(jnp.int32)
                    f = jnp.maximum(s - i0.astype(jnp.float32), 0.0)
                    i1 = jnp.minimum(i0 + 1, R - 1)
                    a0 = plsc.load_gather(lut_v, [zero, i0])
                    a1 = plsc.load_gather(lut_v, [zero, i1])
                    res = a0 + f * (a1 - a0)
                    res = jnp.minimum(jnp.maximum(res, 0.0), 1.0)
                    out_v[0, pl.ds(c + u * _LANES, _LANES)] = res

        pltpu.emit_pipeline(
            body,
            grid=(P * nch,),
            in_specs=[
                pl.BlockSpec((1, _CHUNK), index_map=lambda i: (i // nch, i % nch)),
                pl.BlockSpec((1, R), index_map=lambda i: (i // nch, 0)),
            ],
            out_specs=[
                pl.BlockSpec((1, _CHUNK), index_map=lambda i: (i // nch, i % nch)),
            ],
            core_axis_name=("c", "s"),
            dimension_semantics=(pltpu.PARALLEL,),
        )(imgs_hbm, lut_hbm, out_hbm)

    return run


def kernel(imgs, xform_params):
    N, C, H, W = imgs.shape
    R = xform_params.shape[1]
    P, PIX = N * C, H * W
    imgs2d = imgs.reshape(P, PIX)
    lut = jnp.transpose(xform_params, (0, 2, 1)).reshape(P, R)
    out2d = _make_sc_call(P, PIX, R)(imgs2d, lut)
    return out2d.reshape(N, C, H, W)


# speedup vs baseline: 1260.6300x; 2.9404x over previous
import functools

import jax
import jax.numpy as jnp
from jax.experimental import pallas as pl
from jax.experimental.pallas import tpu as pltpu
from jax.experimental.pallas import tpu_sc as plsc

_LANES = 16
_CHUNK = 16384
_UNROLL = 4


def _make_sc_call(P, PIX, R):
    nch = PIX // _CHUNK
    scale = jnp.float32(R - 1)
    mesh = plsc.VectorSubcoreMesh(core_axis_name="c", subcore_axis_name="s")

    @functools.partial(
        pl.kernel,
        out_type=jax.ShapeDtypeStruct((P, PIX), jnp.float32),
        mesh=mesh,
        compiler_params=pltpu.CompilerParams(needs_layout_passes=False),
    )
    def run(imgs_hbm, lut_hbm, out_hbm):
        def body(in_v, lut_v, out_v):
            zero = jnp.zeros((_LANES,), jnp.int32)

            @plsc.parallel_loop(0, _CHUNK, step=_LANES, unroll=_UNROLL)
            def _(c):
                x = in_v[0, pl.ds(c, _LANES)]
                s = x * scale
                sc = jnp.minimum(jnp.maximum(s, 0.0), scale)
                i0 = sc.astype(jnp.int32)
                f = jnp.maximum(s - i0.astype(jnp.float32), 0.0)
                i1 = jnp.minimum(i0 + 1, R - 1)
                a0 = plsc.load_gather(lut_v, [zero, i0])
                a1 = plsc.load_gather(lut_v, [zero, i1])
                res = a0 + f * (a1 - a0)
                res = jnp.minimum(jnp.maximum(res, 0.0), 1.0)
                out_v[0, pl.ds(c, _LANES)] = res

        pltpu.emit_pipeline(
            body,
            grid=(P * nch,),
            in_specs=[
                pl.BlockSpec((1, _CHUNK), index_map=lambda i: (i // nch, i % nch)),
                pl.BlockSpec((1, R), index_map=lambda i: (i // nch, 0)),
            ],
            out_specs=[
                pl.BlockSpec((1, _CHUNK), index_map=lambda i: (i // nch, i % nch)),
            ],
            core_axis_name=("c", "s"),
            dimension_semantics=(pltpu.PARALLEL,),
        )(imgs_hbm, lut_hbm, out_hbm)

    return run


def kernel(imgs, xform_params):
    N, C, H, W = imgs.shape
    R = xform_params.shape[1]
    P, PIX = N * C, H * W
    imgs2d = imgs.reshape(P, PIX)
    lut = jnp.transpose(xform_params, (0, 2, 1)).reshape(P, R)
    out2d = _make_sc_call(P, PIX, R)(imgs2d, lut)
    return out2d.reshape(N, C, H, W)


# speedup vs baseline: 1320.7745x; 1.0477x over previous
import functools

import jax
import jax.numpy as jnp
from jax.experimental import pallas as pl
from jax.experimental.pallas import tpu as pltpu
from jax.experimental.pallas import tpu_sc as plsc

_LANES = 16
_CHUNK = 16384
_UNROLL = 8


def _make_sc_call(P, PIX, R):
    nch = PIX // _CHUNK
    scale = jnp.float32(R - 1)
    mesh = plsc.VectorSubcoreMesh(core_axis_name="c", subcore_axis_name="s")

    @functools.partial(
        pl.kernel,
        out_type=jax.ShapeDtypeStruct((P, PIX), jnp.float32),
        mesh=mesh,
        compiler_params=pltpu.CompilerParams(needs_layout_passes=False),
    )
    def run(imgs_hbm, lut_hbm, out_hbm):
        def body(in_v, lut_v, out_v):
            zero = jnp.zeros((_LANES,), jnp.int32)

            @plsc.parallel_loop(0, _CHUNK, step=_LANES, unroll=_UNROLL)
            def _(c):
                x = in_v[0, pl.ds(c, _LANES)]
                s = x * scale
                sc = jnp.minimum(jnp.maximum(s, 0.0), scale)
                i0 = sc.astype(jnp.int32)
                f = jnp.maximum(s - i0.astype(jnp.float32), 0.0)
                i1 = jnp.minimum(i0 + 1, R - 1)
                a0 = plsc.load_gather(lut_v, [zero, i0])
                a1 = plsc.load_gather(lut_v, [zero, i1])
                res = a0 + f * (a1 - a0)
                res = jnp.minimum(jnp.maximum(res, 0.0), 1.0)
                out_v[0, pl.ds(c, _LANES)] = res

        pltpu.emit_pipeline(
            body,
            grid=(P * nch,),
            in_specs=[
                pl.BlockSpec((1, _CHUNK), index_map=lambda i: (i // nch, i % nch)),
                pl.BlockSpec((1, R), index_map=lambda i: (i // nch, 0)),
            ],
            out_specs=[
                pl.BlockSpec((1, _CHUNK), index_map=lambda i: (i // nch, i % nch)),
            ],
            core_axis_name=("c", "s"),
            dimension_semantics=(pltpu.PARALLEL,),
        )(imgs_hbm, lut_hbm, out_hbm)

    return run


def kernel(imgs, xform_params):
    N, C, H, W = imgs.shape
    R = xform_params.shape[1]
    P, PIX = N * C, H * W
    imgs2d = imgs.reshape(P, PIX)
    lut = jnp.transpose(xform_params, (0, 2, 1)).reshape(P, R)
    out2d = _make_sc_call(P, PIX, R)(imgs2d, lut)
    return out2d.reshape(N, C, H, W)


# speedup vs baseline: 2415.4083x; 1.8288x over previous
import functools

import jax
import jax.numpy as jnp
from jax.experimental import pallas as pl
from jax.experimental.pallas import tpu as pltpu
from jax.experimental.pallas import tpu_sc as plsc

_LANES = 16
_ROWS = 32
_UNROLL = 8


def _make_sc_call(N, C, H, W, R):
    nch = H // _ROWS
    scale = jnp.float32(R - 1)
    mesh = plsc.VectorSubcoreMesh(core_axis_name="c", subcore_axis_name="s")

    @functools.partial(
        pl.kernel,
        out_type=jax.ShapeDtypeStruct((N, C, H, W), jnp.float32),
        mesh=mesh,
        compiler_params=pltpu.CompilerParams(needs_layout_passes=False),
    )
    def run(imgs_hbm, lut_hbm, out_hbm):
        def body(in_v, lut_v, out_v):
            zero = jnp.zeros((_LANES,), jnp.int32)

            @pl.loop(0, _ROWS)
            def _(r):
                @plsc.parallel_loop(0, W, step=_LANES, unroll=_UNROLL)
                def _(c):
                    x = in_v[0, 0, r, pl.ds(c, _LANES)]
                    s = x * scale
                    sc = jnp.minimum(jnp.maximum(s, 0.0), scale)
                    i0 = sc.astype(jnp.int32)
                    f = jnp.maximum(s - i0.astype(jnp.float32), 0.0)
                    i1 = jnp.minimum(i0 + 1, R - 1)
                    a0 = plsc.load_gather(lut_v, [zero, i0])
                    a1 = plsc.load_gather(lut_v, [zero, i1])
                    res = a0 + f * (a1 - a0)
                    res = jnp.minimum(jnp.maximum(res, 0.0), 1.0)
                    out_v[0, 0, r, pl.ds(c, _LANES)] = res

        pltpu.emit_pipeline(
            body,
            grid=(N * C * nch,),
            in_specs=[
                pl.BlockSpec(
                    (1, 1, _ROWS, W),
                    index_map=lambda i: (i // (C * nch), (i // nch) % C, i % nch, 0),
                ),
                pl.BlockSpec((1, R), index_map=lambda i: (i // nch, 0)),
            ],
            out_specs=[
                pl.BlockSpec(
                    (1, 1, _ROWS, W),
                    index_map=lambda i: (i // (C * nch), (i // nch) % C, i % nch, 0),
                ),
            ],
            core_axis_name=("c", "s"),
            dimension_semantics=(pltpu.PARALLEL,),
        )(imgs_hbm, lut_hbm, out_hbm)

    return run


def kernel(imgs, xform_params):
    N, C, H, W = imgs.shape
    R = xform_params.shape[1]
    lut = jnp.transpose(xform_params, (0, 2, 1)).reshape(N * C, R)
    return _make_sc_call(N, C, H, W, R)(imgs, lut)


# speedup vs baseline: 2693.9651x; 1.1153x over previous
import functools

import jax
import jax.numpy as jnp
from jax.experimental import pallas as pl
from jax.experimental.pallas import tpu as pltpu
from jax.experimental.pallas import tpu_sc as plsc

_LANES = 16
_ROWS = 32
_UNROLL = 8


def _make_sc_call(N, C, H, W, R):
    nch = H // _ROWS
    scale = jnp.float32(R - 1)
    mesh = plsc.VectorSubcoreMesh(core_axis_name="c", subcore_axis_name="s")

    @functools.partial(
        pl.kernel,
        out_type=jax.ShapeDtypeStruct((N, C, H, W), jnp.float32),
        mesh=mesh,
        compiler_params=pltpu.CompilerParams(needs_layout_passes=False),
    )
    def run(imgs_hbm, lut_hbm, out_hbm):
        def body(in_v, lut_v, out_v):
            zero = jnp.zeros((_LANES,), jnp.int32)

            @plsc.parallel_loop(0, _ROWS * W, step=_LANES, unroll=_UNROLL)
            def _(flat):
                r = flat // W
                c = flat % W
                x = in_v[0, 0, r, pl.ds(c, _LANES)]
                s = x * scale
                sc = jnp.minimum(jnp.maximum(s, 0.0), scale)
                i0 = sc.astype(jnp.int32)
                f = jnp.maximum(s - i0.astype(jnp.float32), 0.0)
                i1 = jnp.minimum(i0 + 1, R - 1)
                a0 = plsc.load_gather(lut_v, [zero, i0])
                a1 = plsc.load_gather(lut_v, [zero, i1])
                res = a0 + f * (a1 - a0)
                res = jnp.minimum(jnp.maximum(res, 0.0), 1.0)
                out_v[0, 0, r, pl.ds(c, _LANES)] = res

        pltpu.emit_pipeline(
            body,
            grid=(N * C * nch,),
            in_specs=[
                pl.BlockSpec(
                    (1, 1, _ROWS, W),
                    index_map=lambda i: (i // (C * nch), (i // nch) % C, i % nch, 0),
                ),
                pl.BlockSpec((1, R), index_map=lambda i: (i // nch, 0)),
            ],
            out_specs=[
                pl.BlockSpec(
                    (1, 1, _ROWS, W),
                    index_map=lambda i: (i // (C * nch), (i // nch) % C, i % nch, 0),
                ),
            ],
            core_axis_name=("c", "s"),
            dimension_semantics=(pltpu.PARALLEL,),
        )(imgs_hbm, lut_hbm, out_hbm)

    return run


def kernel(imgs, xform_params):
    N, C, H, W = imgs.shape
    R = xform_params.shape[1]
    lut = jnp.transpose(xform_params, (0, 2, 1)).reshape(N * C, R)
    return _make_sc_call(N, C, H, W, R)(imgs, lut)
